# Initial kernel scaffold; baseline (speedup 1.0000x reference)
import jax
import jax.numpy as jnp
from jax.experimental import pallas as pl


def kernel(features, edge_index, W1, b1, W2, b2):
    raise NotImplementedError("write your pallas kernel here")


# speedup vs baseline: 3.1067x; 3.1067x over previous
import functools

import jax
import jax.numpy as jnp
from jax import lax
from jax.experimental import pallas as pl
from jax.experimental.pallas import tpu as pltpu
from jax.experimental.pallas import tpu_sc as plsc

N = 10000
NP = 10240
E = 160000
D = 256
DH = 128
NS = 16
ROWS_PER_TEC = NP // NS
EDGES_PER_TEC = E // NS
CH = 80
NCHUNK = EDGES_PER_TEC // CH
ZR = 128

_MESH = plsc.VectorSubcoreMesh(
    core_axis_name="c", subcore_axis_name="s", num_cores=2, num_subcores=NS)


def _deg_body(src_hbm, dst_hbm, dego_hbm, degi_hbm, deg_sh, idx_v, ones_v,
              zbuf_v):
    c = lax.axis_index("c")
    s = lax.axis_index("s")

    def fill_z(i, carry):
        zbuf_v[i] = jnp.zeros((16,), jnp.float32)
        return carry

    lax.fori_loop(0, ROWS_PER_TEC, fill_z, 0)

    def fill_o(i, carry):
        ones_v[i] = jnp.ones((16,), jnp.float32)
        return carry

    lax.fori_loop(0, CH, fill_o, 0)

    pltpu.sync_copy(zbuf_v, deg_sh.at[pl.ds(s * ROWS_PER_TEC, ROWS_PER_TEC)])
    plsc.subcore_barrier()

    def chunk(k, carry):
        base = s * EDGES_PER_TEC + k * CH

        @pl.when(c == 0)
        def _():
            pltpu.sync_copy(src_hbm.at[pl.ds(base, CH)], idx_v)

        @pl.when(c == 1)
        def _():
            pltpu.sync_copy(dst_hbm.at[pl.ds(base, CH)], idx_v)

        pltpu.sync_copy(ones_v, deg_sh.at[idx_v], add=True)
        return carry

    lax.fori_loop(0, NCHUNK, chunk, 0)
    plsc.subcore_barrier()

    rows = pl.ds(s * ROWS_PER_TEC, ROWS_PER_TEC)

    @pl.when(c == 0)
    def _():
        pltpu.sync_copy(deg_sh.at[rows], dego_hbm.at[rows])

    @pl.when(c == 1)
    def _():
        pltpu.sync_copy(deg_sh.at[rows], degi_hbm.at[rows])


_deg_kernel = pl.kernel(
    _deg_body,
    out_type=(
        jax.ShapeDtypeStruct((NP, 16), jnp.float32),
        jax.ShapeDtypeStruct((NP, 16), jnp.float32),
    ),
    mesh=_MESH,
    scratch_types=[
        pltpu.VMEM_SHARED((NP, 16), jnp.float32),
        pltpu.VMEM((CH,), jnp.int32),
        pltpu.VMEM((CH, 16), jnp.float32),
        pltpu.VMEM((ROWS_PER_TEC, 16), jnp.float32),
    ],
)


def _agg_body(m_lo, m_hi, src_hbm, dst_hbm, out_lo, out_hi, acc_sh, sidx_v,
              didx_v, rows_v, zbuf_v, sem):
    c = lax.axis_index("c")
    s = lax.axis_index("s")

    def fill_z(i, carry):
        for j in range(DH // 16):
            zbuf_v[i, pl.ds(j * 16, 16)] = jnp.zeros((16,), jnp.float32)
        return carry

    lax.fori_loop(0, ZR, fill_z, 0)

    for part in range(ROWS_PER_TEC // ZR):
        pltpu.sync_copy(
            zbuf_v, acc_sh.at[pl.ds(s * ROWS_PER_TEC + part * ZR, ZR)])
    plsc.subcore_barrier()

    def chunk(k, carry):
        base = s * EDGES_PER_TEC + k * CH
        pltpu.sync_copy(src_hbm.at[pl.ds(base, CH)], sidx_v)
        pltpu.sync_copy(dst_hbm.at[pl.ds(base, CH)], didx_v)

        @pl.when(c == 0)
        def _():
            pltpu.async_copy(m_lo.at[sidx_v], rows_v, sem).wait()

        @pl.when(c == 1)
        def _():
            pltpu.async_copy(m_hi.at[sidx_v], rows_v, sem).wait()

        pltpu.sync_copy(rows_v, acc_sh.at[didx_v], add=True)
        return carry

    lax.fori_loop(0, NCHUNK, chunk, 0)
    plsc.subcore_barrier()

    rows = pl.ds(s * ROWS_PER_TEC, ROWS_PER_TEC)

    @pl.when(c == 0)
    def _():
        pltpu.sync_copy(acc_sh.at[rows], out_lo.at[rows])

    @pl.when(c == 1)
    def _():
        pltpu.sync_copy(acc_sh.at[rows], out_hi.at[rows])


_agg_kernel = pl.kernel(
    _agg_body,
    out_type=(
        jax.ShapeDtypeStruct((NP, DH), jnp.float32),
        jax.ShapeDtypeStruct((NP, DH), jnp.float32),
    ),
    mesh=_MESH,
    scratch_types=[
        pltpu.VMEM_SHARED((NP, DH), jnp.float32),
        pltpu.VMEM((CH,), jnp.int32),
        pltpu.VMEM((CH,), jnp.int32),
        pltpu.VMEM((CH, DH), jnp.float32),
        pltpu.VMEM((ZR, DH), jnp.float32),
        pltpu.SemaphoreType.DMA,
    ],
)


TM = 256
GRID_M = NP // TM


def _norm_col(deg_block):
    d = deg_block[:, 0:1]
    return jnp.where(d > 0, lax.rsqrt(d), 0.0)


def _mm1_body(f_ref, w_ref, dego_ref, lo_ref, hi_ref):
    no = _norm_col(dego_ref)
    x = f_ref[...] * no
    y = jnp.dot(x, w_ref[...], preferred_element_type=jnp.float32)
    lo_ref[...] = y[:, :DH]
    hi_ref[...] = y[:, DH:]


_mm1 = pl.pallas_call(
    _mm1_body,
    grid=(GRID_M,),
    in_specs=[
        pl.BlockSpec((TM, D), lambda i: (i, 0)),
        pl.BlockSpec((D, D), lambda i: (0, 0)),
        pl.BlockSpec((TM, 16), lambda i: (i, 0)),
    ],
    out_specs=[
        pl.BlockSpec((TM, DH), lambda i: (i, 0)),
        pl.BlockSpec((TM, DH), lambda i: (i, 0)),
    ],
    out_shape=(
        jax.ShapeDtypeStruct((NP, DH), jnp.float32),
        jax.ShapeDtypeStruct((NP, DH), jnp.float32),
    ),
)


def _mid_body(glo_ref, ghi_ref, degi_ref, dego_ref, b_ref, w_ref, lo_ref,
              hi_ref):
    ni = _norm_col(degi_ref)
    no = _norm_col(dego_ref)
    b = b_ref[...]
    h_lo = jnp.maximum(glo_ref[...] * ni + b[0, :DH], 0.0) * no
    h_hi = jnp.maximum(ghi_ref[...] * ni + b[0, DH:], 0.0) * no
    w = w_ref[...]
    y = (jnp.dot(h_lo, w[:DH, :], preferred_element_type=jnp.float32) +
         jnp.dot(h_hi, w[DH:, :], preferred_element_type=jnp.float32))
    lo_ref[...] = y[:, :DH]
    hi_ref[...] = y[:, DH:]


_mid = pl.pallas_call(
    _mid_body,
    grid=(GRID_M,),
    in_specs=[
        pl.BlockSpec((TM, DH), lambda i: (i, 0)),
        pl.BlockSpec((TM, DH), lambda i: (i, 0)),
        pl.BlockSpec((TM, 16), lambda i: (i, 0)),
        pl.BlockSpec((TM, 16), lambda i: (i, 0)),
        pl.BlockSpec((1, D), lambda i: (0, 0)),
        pl.BlockSpec((D, D), lambda i: (0, 0)),
    ],
    out_specs=[
        pl.BlockSpec((TM, DH), lambda i: (i, 0)),
        pl.BlockSpec((TM, DH), lambda i: (i, 0)),
    ],
    out_shape=(
        jax.ShapeDtypeStruct((NP, DH), jnp.float32),
        jax.ShapeDtypeStruct((NP, DH), jnp.float32),
    ),
)


def _fin_body(glo_ref, ghi_ref, degi_ref, b_ref, out_ref):
    ni = _norm_col(degi_ref)
    b = b_ref[...]
    out_ref[:, :DH] = glo_ref[...] * ni + b[0, :DH]
    out_ref[:, DH:] = ghi_ref[...] * ni + b[0, DH:]


_fin = pl.pallas_call(
    _fin_body,
    grid=(GRID_M,),
    in_specs=[
        pl.BlockSpec((TM, DH), lambda i: (i, 0)),
        pl.BlockSpec((TM, DH), lambda i: (i, 0)),
        pl.BlockSpec((TM, 16), lambda i: (i, 0)),
        pl.BlockSpec((1, D), lambda i: (0, 0)),
    ],
    out_specs=pl.BlockSpec((TM, D), lambda i: (i, 0)),
    out_shape=jax.ShapeDtypeStruct((N, D), jnp.float32),
)


@jax.jit
def kernel(features, edge_index, W1, b1, W2, b2):
    src = edge_index[0]
    dst = edge_index[1]
    dego, degi = _deg_kernel(src, dst)
    m1_lo, m1_hi = _mm1(features, W1, dego)
    g1_lo, g1_hi = _agg_kernel(m1_lo, m1_hi, src, dst)
    m2_lo, m2_hi = _mid(g1_lo, g1_hi, degi, dego, b1.reshape(1, D), W2)
    g2_lo, g2_hi = _agg_kernel(m2_lo, m2_hi, src, dst)
    return _fin(g2_lo, g2_hi, degi, b2.reshape(1, D))
